# baseline (speedup 1.0000x reference)
import jax
import jax.numpy as jnp
from jax import lax
from jax.experimental import pallas as pl
from jax.experimental.pallas import tpu as pltpu
from jax.experimental.pallas import tpu_sc as plsc

N = 10000
E = 320000
D = 128
NC = 2
NS = 16
NW = NC * NS
EPW = E // NW
CH = 80
NCHUNK = EPW // CH
LN = 16
HA = 48
HB = CH - HA


def _pre_body(x_ref, wg_ref, al_ref, ar_ref, z_ref, el_ref, er_ref, m_ref):
    z = lax.dot_general(x_ref[...], wg_ref[...], (((1,), (1,)), ((), ())),
                        preferred_element_type=jnp.float32,
                        precision=lax.Precision.HIGHEST)
    z_ref[...] = z
    el = jnp.sum(z * al_ref[...][None, :], axis=1)
    er = jnp.sum(z * ar_ref[...][None, :], axis=1)
    el_ref[...] = el
    er_ref[...] = er
    m = jnp.max(el) + jnp.max(er)
    m = jnp.where(m >= 0.0, m, 0.2 * m)
    m_ref[...] = jnp.full((LN,), m, jnp.float32)


def _post_body(ph_ref, pd_ref, b_ref, wl_ref, o_ref):
    hu = ph_ref[0] + ph_ref[1]
    den = pd_ref[0, :, 0:1] + pd_ref[1, :, 0:1]
    h = hu / (den + 1e-9) + b_ref[...][None, :]
    h = jnp.where(h >= 0.0, h, 0.01 * h)
    o_ref[...] = lax.dot_general(h, wl_ref[...], (((1,), (1,)), ((), ())),
                                 preferred_element_type=jnp.float32,
                                 precision=lax.Precision.HIGHEST)


def _edge_body(z_hbm, el_hbm, er_hbm, sd_hbm, m_hbm,
               outh_hbm, outd_hbm,
               el_v, er_v, m_v, sdA, sdB, zrowA, zrowB, wrowA, wrowB,
               dstA, dstB, w_v,
               acch_sh, accd_sh, gsemA, gsemB, hsemA, hsemB, dsemA, dsemB,
               isem):
    cid = lax.axis_index("c")
    sid = lax.axis_index("s")
    wid = cid * NS + sid

    pltpu.sync_copy(el_hbm, el_v)
    pltpu.sync_copy(er_hbm, er_v)
    pltpu.sync_copy(m_hbm, m_v)
    mvec = m_v[...]
    lane = lax.iota(jnp.int32, LN)
    zero16 = jnp.zeros((LN,), jnp.float32)

    @pl.loop(0, HA)
    def _zeroA(j):
        for c in range(D // LN):
            zrowA[j, pl.ds(c * LN, LN)] = zero16
        wrowA[j, :] = zero16

    @pl.loop(0, HB)
    def _zeroB(j):
        for c in range(D // LN):
            zrowB[j, pl.ds(c * LN, LN)] = zero16
        wrowB[j, :] = zero16

    @pl.loop(sid, N // CH, step=NS)
    def _zcopy(g):
        pltpu.sync_copy(zrowA, acch_sh.at[pl.ds(g * CH, HA)])
        pltpu.sync_copy(zrowB, acch_sh.at[pl.ds(g * CH + HA, HB)])
        pltpu.sync_copy(wrowA, accd_sh.at[pl.ds(g * CH, HA)])
        pltpu.sync_copy(wrowB, accd_sh.at[pl.ds(g * CH + HA, HB)])

    plsc.subcore_barrier()

    cbase = wid * NCHUNK

    def scale_rows(zrow, nrows, wof):
        @plsc.parallel_loop(0, nrows, unroll=2)
        def _scale(j):
            wv = jnp.full((LN,), w_v[pl.ds(wof + j, LN)][0], jnp.float32)
            for c in range(D // LN):
                zrow[j, pl.ds(c * LN, LN)] = zrow[j, pl.ds(c * LN, LN)] * wv

    zcol = jnp.zeros((LN,), jnp.int32)

    def chunk_body(ci, sdP, sdQ, prev_pred, has_next):
        if has_next:
            pltpu.async_copy(sd_hbm.at[cbase + ci + 1], sdQ, isem)

        for g in range(CH // LN):
            si = sdP[0, pl.ds(g * LN, LN)]
            di = sdP[1, pl.ds(g * LN, LN)]
            s = plsc.load_gather(el_v, [si]) + plsc.load_gather(er_v, [di])
            e = jnp.where(s >= 0.0, s, 0.2 * s)
            w_v[pl.ds(g * LN, LN)] = jnp.exp(e - mvec)

        def drain_prev_b():
            pltpu.make_async_copy(zrowB, acch_sh.at[dstB], hsemB).wait()
            pltpu.make_async_copy(wrowB, accd_sh.at[dstB], dsemB).wait()

        if prev_pred is None:
            drain_prev_b()
        else:
            pl.when(prev_pred)(drain_prev_b)

        for g in range(HA // LN):
            dstA[pl.ds(g * LN, LN)] = sdP[1, pl.ds(g * LN, LN)]
            plsc.store_scatter(wrowA, [lane + g * LN, zcol],
                               w_v[pl.ds(g * LN, LN)])
        for g in range(HB // LN):
            dstB[pl.ds(g * LN, LN)] = sdP[1, pl.ds(HA + g * LN, LN)]
            plsc.store_scatter(wrowB, [lane + g * LN, zcol],
                               w_v[pl.ds(HA + g * LN, LN)])

        pltpu.make_async_copy(z_hbm.at[sdP.at[0, pl.ds(0, HA)]],
                              zrowA, gsemA).wait()
        pltpu.async_copy(z_hbm.at[sdP.at[0, pl.ds(HA, HB)]], zrowB, gsemB)

        scale_rows(zrowA, HA, 0)
        pltpu.async_copy(zrowA, acch_sh.at[dstA], hsemA, add=True)
        pltpu.async_copy(wrowA, accd_sh.at[dstA], dsemA, add=True)

        pltpu.make_async_copy(z_hbm.at[sdP.at[0, pl.ds(HA, HB)]],
                              zrowB, gsemB).wait()
        scale_rows(zrowB, HB, HA)

        pltpu.make_async_copy(zrowA, acch_sh.at[dstA], hsemA).wait()
        pltpu.make_async_copy(wrowA, accd_sh.at[dstA], dsemA).wait()
        if has_next:
            pltpu.make_async_copy(sd_hbm.at[cbase + ci + 1], sdQ, isem).wait()
            pltpu.async_copy(z_hbm.at[sdQ.at[0, pl.ds(0, HA)]], zrowA, gsemA)

        pltpu.async_copy(zrowB, acch_sh.at[dstB], hsemB, add=True)
        pltpu.async_copy(wrowB, accd_sh.at[dstB], dsemB, add=True)

    pltpu.sync_copy(sd_hbm.at[cbase], sdA)
    pltpu.async_copy(z_hbm.at[sdA.at[0, pl.ds(0, HA)]], zrowA, gsemA)

    @pl.loop(0, NCHUNK // 2)
    def _pair(k):
        chunk_body(2 * k, sdA, sdB, k > 0, True)
        chunk_body(2 * k + 1, sdB, sdA, None, True)

    chunk_body(NCHUNK - 1, sdA, sdB, None, False)
    pltpu.make_async_copy(zrowB, acch_sh.at[dstB], hsemB).wait()
    pltpu.make_async_copy(wrowB, accd_sh.at[dstB], dsemB).wait()

    plsc.subcore_barrier()

    @pl.loop(sid, N // CH, step=NS)
    def _out(g):
        pltpu.sync_copy(acch_sh.at[pl.ds(g * CH, CH)],
                        outh_hbm.at[cid, pl.ds(g * CH, CH)])
        pltpu.sync_copy(accd_sh.at[pl.ds(g * CH, CH)],
                        outd_hbm.at[cid, pl.ds(g * CH, CH)])


def kernel(x, edge_index, W_gat, attn_l, attn_r, bias_gat, W_lin):
    ei = edge_index.astype(jnp.int32)
    sd = ei.reshape(2, E // CH, CH).transpose(1, 0, 2)

    z, el, er, m = pl.pallas_call(
        _pre_body,
        out_shape=[
            jax.ShapeDtypeStruct((N, D), jnp.float32),
            jax.ShapeDtypeStruct((N,), jnp.float32),
            jax.ShapeDtypeStruct((N,), jnp.float32),
            jax.ShapeDtypeStruct((LN,), jnp.float32),
        ],
    )(x, W_gat, attn_l, attn_r)

    mesh = plsc.VectorSubcoreMesh(core_axis_name="c", subcore_axis_name="s",
                                  num_cores=NC, num_subcores=NS)
    edge_kernel = pl.kernel(
        _edge_body,
        out_type=[
            jax.ShapeDtypeStruct((NC, N, D), jnp.float32),
            jax.ShapeDtypeStruct((NC, N, LN), jnp.float32),
        ],
        mesh=mesh,
        compiler_params=pltpu.CompilerParams(use_tc_tiling_on_sc=False,
                                             needs_layout_passes=False),
        scratch_types=[
            pltpu.VMEM((N,), jnp.float32),
            pltpu.VMEM((N,), jnp.float32),
            pltpu.VMEM((LN,), jnp.float32),
            pltpu.VMEM((2, CH), jnp.int32),
            pltpu.VMEM((2, CH), jnp.int32),
            pltpu.VMEM((HA, D), jnp.float32),
            pltpu.VMEM((HB, D), jnp.float32),
            pltpu.VMEM((HA, LN), jnp.float32),
            pltpu.VMEM((HB, LN), jnp.float32),
            pltpu.VMEM((HA,), jnp.int32),
            pltpu.VMEM((HB,), jnp.int32),
            pltpu.VMEM((CH + LN,), jnp.float32),
            pltpu.VMEM_SHARED((N, D), jnp.float32),
            pltpu.VMEM_SHARED((N, LN), jnp.float32),
            pltpu.SemaphoreType.DMA,
            pltpu.SemaphoreType.DMA,
            pltpu.SemaphoreType.DMA,
            pltpu.SemaphoreType.DMA,
            pltpu.SemaphoreType.DMA,
            pltpu.SemaphoreType.DMA,
            pltpu.SemaphoreType.DMA,
        ],
    )
    parts_h, parts_d = edge_kernel(z, el, er, sd, m)

    out = pl.pallas_call(
        _post_body,
        out_shape=jax.ShapeDtypeStruct((N, D), jnp.float32),
    )(parts_h, parts_d, bias_gat, W_lin)
    return out


# speedup vs baseline: 1.1499x; 1.1499x over previous
import jax
import jax.numpy as jnp
from jax import lax
from jax.experimental import pallas as pl
from jax.experimental.pallas import tpu as pltpu
from jax.experimental.pallas import tpu_sc as plsc

N = 10000
E = 320000
D = 128
NC = 2
NS = 16
NW = NC * NS
EPW = E // NW
CH = 80
NCHUNK = EPW // CH
LN = 16
HA = 48
HB = CH - HA


def _pre_body(x_ref, wg_ref, al_ref, ar_ref, z_ref, el_ref, er_ref, m_ref):
    z = lax.dot_general(x_ref[...], wg_ref[...], (((1,), (1,)), ((), ())),
                        preferred_element_type=jnp.float32,
                        precision=lax.Precision.HIGHEST)
    z_ref[...] = z
    el = jnp.sum(z * al_ref[...][None, :], axis=1)
    er = jnp.sum(z * ar_ref[...][None, :], axis=1)
    el_ref[...] = el
    er_ref[...] = er
    m = jnp.max(el) + jnp.max(er)
    m = jnp.where(m >= 0.0, m, 0.2 * m)
    m_ref[...] = jnp.full((LN,), m, jnp.float32)


def _post_body(ph_ref, pd_ref, b_ref, wl_ref, o_ref):
    hu = ph_ref[0] + ph_ref[1]
    den = pd_ref[0, :, 0:1] + pd_ref[1, :, 0:1]
    h = hu / (den + 1e-9) + b_ref[...][None, :]
    h = jnp.where(h >= 0.0, h, 0.01 * h)
    o_ref[...] = lax.dot_general(h, wl_ref[...], (((1,), (1,)), ((), ())),
                                 preferred_element_type=jnp.float32,
                                 precision=lax.Precision.HIGHEST)


def _edge_body(z_hbm, el_hbm, er_hbm, sd_hbm, m_hbm,
               outh_hbm, outd_hbm,
               el_v, er_v, m_v, sdA, sdB, zrowA, zrowB, wrowA, wrowB,
               dstA, dstB, w_v,
               acch_sh, accd_sh, gsemA, gsemB, hsemA, hsemB, dsemA, dsemB,
               isem):
    cid = lax.axis_index("c")
    sid = lax.axis_index("s")
    wid = cid * NS + sid

    pltpu.sync_copy(el_hbm, el_v)
    pltpu.sync_copy(er_hbm, er_v)
    pltpu.sync_copy(m_hbm, m_v)
    mvec = m_v[...]
    lane = lax.iota(jnp.int32, LN)
    zero16 = jnp.zeros((LN,), jnp.float32)

    @pl.loop(0, HA)
    def _zeroA(j):
        for c in range(D // LN):
            zrowA[j, pl.ds(c * LN, LN)] = zero16
        wrowA[j, :] = zero16

    @pl.loop(0, HB)
    def _zeroB(j):
        for c in range(D // LN):
            zrowB[j, pl.ds(c * LN, LN)] = zero16
        wrowB[j, :] = zero16

    @pl.loop(sid, N // CH, step=NS)
    def _zcopy(g):
        pltpu.sync_copy(zrowA, acch_sh.at[pl.ds(g * CH, HA)])
        pltpu.sync_copy(zrowB, acch_sh.at[pl.ds(g * CH + HA, HB)])
        pltpu.sync_copy(wrowA, accd_sh.at[pl.ds(g * CH, HA)])
        pltpu.sync_copy(wrowB, accd_sh.at[pl.ds(g * CH + HA, HB)])

    plsc.subcore_barrier()

    cbase = wid * NCHUNK

    def scale_rows(zrow, nrows, wof):
        @plsc.parallel_loop(0, nrows, unroll=2)
        def _scale(j):
            wv = jnp.full((LN,), w_v[pl.ds(wof + j, LN)][0], jnp.float32)
            for c in range(D // LN):
                zrow[j, pl.ds(c * LN, LN)] = zrow[j, pl.ds(c * LN, LN)] * wv

    zcol = jnp.zeros((LN,), jnp.int32)

    def chunk_body(ci, sdP, sdQ, prev_pred, has_next):
        if has_next:
            pltpu.async_copy(sd_hbm.at[cbase + ci + 1], sdQ, isem)

        for g in range(CH // LN):
            si = sdP[0, pl.ds(g * LN, LN)]
            di = sdP[1, pl.ds(g * LN, LN)]
            s = plsc.load_gather(el_v, [si]) + plsc.load_gather(er_v, [di])
            e = jnp.where(s >= 0.0, s, 0.2 * s)
            w_v[pl.ds(g * LN, LN)] = jnp.exp(e - mvec)

        def drain_prev_b():
            pltpu.make_async_copy(zrowB, acch_sh.at[dstB], hsemB).wait()
            pltpu.make_async_copy(wrowB, accd_sh.at[dstB], dsemB).wait()

        if prev_pred is None:
            drain_prev_b()
        else:
            pl.when(prev_pred)(drain_prev_b)

        for g in range(HA // LN):
            dstA[pl.ds(g * LN, LN)] = sdP[1, pl.ds(g * LN, LN)]
            plsc.store_scatter(wrowA, [lane + g * LN, zcol],
                               w_v[pl.ds(g * LN, LN)])
        for g in range(HB // LN):
            dstB[pl.ds(g * LN, LN)] = sdP[1, pl.ds(HA + g * LN, LN)]
            plsc.store_scatter(wrowB, [lane + g * LN, zcol],
                               w_v[pl.ds(HA + g * LN, LN)])

        pltpu.async_copy(z_hbm.at[sdP.at[0, pl.ds(HA, HB)]], zrowB, gsemB)
        pltpu.make_async_copy(z_hbm.at[sdP.at[0, pl.ds(0, HA)]],
                              zrowA, gsemA).wait()

        scale_rows(zrowA, HA, 0)
        pltpu.async_copy(zrowA, acch_sh.at[dstA], hsemA, add=True)
        pltpu.async_copy(wrowA, accd_sh.at[dstA], dsemA, add=True)

        pltpu.make_async_copy(z_hbm.at[sdP.at[0, pl.ds(HA, HB)]],
                              zrowB, gsemB).wait()
        scale_rows(zrowB, HB, HA)

        pltpu.make_async_copy(zrowA, acch_sh.at[dstA], hsemA).wait()
        pltpu.make_async_copy(wrowA, accd_sh.at[dstA], dsemA).wait()
        if has_next:
            pltpu.make_async_copy(sd_hbm.at[cbase + ci + 1], sdQ, isem).wait()
            pltpu.async_copy(z_hbm.at[sdQ.at[0, pl.ds(0, HA)]], zrowA, gsemA)

        pltpu.async_copy(zrowB, acch_sh.at[dstB], hsemB, add=True)
        pltpu.async_copy(wrowB, accd_sh.at[dstB], dsemB, add=True)

    pltpu.sync_copy(sd_hbm.at[cbase], sdA)
    pltpu.async_copy(z_hbm.at[sdA.at[0, pl.ds(0, HA)]], zrowA, gsemA)

    @pl.loop(0, NCHUNK // 2)
    def _pair(k):
        chunk_body(2 * k, sdA, sdB, k > 0, True)
        chunk_body(2 * k + 1, sdB, sdA, None, True)

    chunk_body(NCHUNK - 1, sdA, sdB, None, False)
    pltpu.make_async_copy(zrowB, acch_sh.at[dstB], hsemB).wait()
    pltpu.make_async_copy(wrowB, accd_sh.at[dstB], dsemB).wait()

    plsc.subcore_barrier()

    @pl.loop(sid, N // CH, step=NS)
    def _out(g):
        pltpu.sync_copy(acch_sh.at[pl.ds(g * CH, CH)],
                        outh_hbm.at[cid, pl.ds(g * CH, CH)])
        pltpu.sync_copy(accd_sh.at[pl.ds(g * CH, CH)],
                        outd_hbm.at[cid, pl.ds(g * CH, CH)])


def kernel(x, edge_index, W_gat, attn_l, attn_r, bias_gat, W_lin):
    ei = edge_index.astype(jnp.int32)
    sd = ei.reshape(2, E // CH, CH).transpose(1, 0, 2)

    z, el, er, m = pl.pallas_call(
        _pre_body,
        out_shape=[
            jax.ShapeDtypeStruct((N, D), jnp.float32),
            jax.ShapeDtypeStruct((N,), jnp.float32),
            jax.ShapeDtypeStruct((N,), jnp.float32),
            jax.ShapeDtypeStruct((LN,), jnp.float32),
        ],
    )(x, W_gat, attn_l, attn_r)

    mesh = plsc.VectorSubcoreMesh(core_axis_name="c", subcore_axis_name="s",
                                  num_cores=NC, num_subcores=NS)
    edge_kernel = pl.kernel(
        _edge_body,
        out_type=[
            jax.ShapeDtypeStruct((NC, N, D), jnp.float32),
            jax.ShapeDtypeStruct((NC, N, LN), jnp.float32),
        ],
        mesh=mesh,
        compiler_params=pltpu.CompilerParams(use_tc_tiling_on_sc=False,
                                             needs_layout_passes=False),
        scratch_types=[
            pltpu.VMEM((N,), jnp.float32),
            pltpu.VMEM((N,), jnp.float32),
            pltpu.VMEM((LN,), jnp.float32),
            pltpu.VMEM((2, CH), jnp.int32),
            pltpu.VMEM((2, CH), jnp.int32),
            pltpu.VMEM((HA, D), jnp.float32),
            pltpu.VMEM((HB, D), jnp.float32),
            pltpu.VMEM((HA, LN), jnp.float32),
            pltpu.VMEM((HB, LN), jnp.float32),
            pltpu.VMEM((HA,), jnp.int32),
            pltpu.VMEM((HB,), jnp.int32),
            pltpu.VMEM((CH + LN,), jnp.float32),
            pltpu.VMEM_SHARED((N, D), jnp.float32),
            pltpu.VMEM_SHARED((N, LN), jnp.float32),
            pltpu.SemaphoreType.DMA,
            pltpu.SemaphoreType.DMA,
            pltpu.SemaphoreType.DMA,
            pltpu.SemaphoreType.DMA,
            pltpu.SemaphoreType.DMA,
            pltpu.SemaphoreType.DMA,
            pltpu.SemaphoreType.DMA,
        ],
    )
    parts_h, parts_d = edge_kernel(z, el, er, sd, m)

    out = pl.pallas_call(
        _post_body,
        out_shape=jax.ShapeDtypeStruct((N, D), jnp.float32),
    )(parts_h, parts_d, bias_gat, W_lin)
    return out


# speedup vs baseline: 1.2140x; 1.0557x over previous
import jax
import jax.numpy as jnp
from jax import lax
from jax.experimental import pallas as pl
from jax.experimental.pallas import tpu as pltpu
from jax.experimental.pallas import tpu_sc as plsc

N = 10000
E = 320000
D = 128
NC = 2
NS = 16
NW = NC * NS
EPW = E // NW
CH = 80
NCHUNK = EPW // CH
LN = 16
HA = 48
HB = CH - HA


def _pre_body(x_ref, wg_ref, al_ref, ar_ref, z_ref, el_ref, er_ref, m_ref):
    z = lax.dot_general(x_ref[...], wg_ref[...], (((1,), (1,)), ((), ())),
                        preferred_element_type=jnp.float32,
                        precision=lax.Precision.HIGHEST)
    z_ref[...] = z
    el = jnp.sum(z * al_ref[...][None, :], axis=1)
    er = jnp.sum(z * ar_ref[...][None, :], axis=1)
    el_ref[...] = el
    er_ref[...] = er
    m = jnp.max(el) + jnp.max(er)
    m = jnp.where(m >= 0.0, m, 0.2 * m)
    m_ref[...] = jnp.full((LN,), m, jnp.float32)


def _post_body(ph_ref, pd_ref, b_ref, wl_ref, o_ref):
    hu = ph_ref[0] + ph_ref[1]
    den = pd_ref[0, :, 0:1] + pd_ref[1, :, 0:1]
    h = hu / (den + 1e-9) + b_ref[...][None, :]
    h = jnp.where(h >= 0.0, h, 0.01 * h)
    o_ref[...] = lax.dot_general(h, wl_ref[...], (((1,), (1,)), ((), ())),
                                 preferred_element_type=jnp.float32,
                                 precision=lax.Precision.HIGHEST)


def _edge_body(z_hbm, el_hbm, er_hbm, sd_hbm, m_hbm,
               outh_hbm, outd_hbm,
               el_v, er_v, m_v, sdA, sdB, zA1, zA2, zrowB,
               wA1, wA2, wrowB, dA1, dA2, dstB, w_v,
               acch_sh, accd_sh, gsemA, gsemB, hA1sem, hA2sem,
               dA1sem, dA2sem, hsemB, dsemB, isem):
    cid = lax.axis_index("c")
    sid = lax.axis_index("s")
    wid = cid * NS + sid

    pltpu.sync_copy(el_hbm, el_v)
    pltpu.sync_copy(er_hbm, er_v)
    pltpu.sync_copy(m_hbm, m_v)
    mvec = m_v[...]
    lane = lax.iota(jnp.int32, LN)
    zero16 = jnp.zeros((LN,), jnp.float32)

    @pl.loop(0, HA)
    def _zeroA(j):
        for c in range(D // LN):
            zA1[j, pl.ds(c * LN, LN)] = zero16
        wA1[j, :] = zero16
        wA2[j, :] = zero16

    @pl.loop(0, HB)
    def _zeroB(j):
        for c in range(D // LN):
            zrowB[j, pl.ds(c * LN, LN)] = zero16
        wrowB[j, :] = zero16

    @pl.loop(sid, N // CH, step=NS)
    def _zcopy(g):
        pltpu.sync_copy(zA1, acch_sh.at[pl.ds(g * CH, HA)])
        pltpu.sync_copy(zrowB, acch_sh.at[pl.ds(g * CH + HA, HB)])
        pltpu.sync_copy(wA1, accd_sh.at[pl.ds(g * CH, HA)])
        pltpu.sync_copy(wrowB, accd_sh.at[pl.ds(g * CH + HA, HB)])

    plsc.subcore_barrier()

    cbase = wid * NCHUNK

    def scale_rows(zrow, nrows, wof):
        @plsc.parallel_loop(0, nrows, unroll=2)
        def _scale(j):
            wv = jnp.full((LN,), w_v[pl.ds(wof + j, LN)][0], jnp.float32)
            for c in range(D // LN):
                zrow[j, pl.ds(c * LN, LN)] = zrow[j, pl.ds(c * LN, LN)] * wv

    zcol = jnp.zeros((LN,), jnp.int32)

    def chunk_body(ci, sdP, sdQ, zaP, waP, daP, hsemP, dsemP,
                   zaQ, waQ, daQ, hsemQ, dsemQ, prev_pred, has_next):
        if has_next:
            pltpu.async_copy(sd_hbm.at[cbase + ci + 1], sdQ, isem)

        for g in range(CH // LN):
            si = sdP[0, pl.ds(g * LN, LN)]
            di = sdP[1, pl.ds(g * LN, LN)]
            s = plsc.load_gather(el_v, [si]) + plsc.load_gather(er_v, [di])
            e = jnp.where(s >= 0.0, s, 0.2 * s)
            w_v[pl.ds(g * LN, LN)] = jnp.exp(e - mvec)

        def drain_prev_b():
            pltpu.make_async_copy(zrowB, acch_sh.at[dstB], hsemB).wait()
            pltpu.make_async_copy(wrowB, accd_sh.at[dstB], dsemB).wait()

        if prev_pred is None:
            drain_prev_b()
        else:
            pl.when(prev_pred)(drain_prev_b)

        for g in range(HA // LN):
            daP[pl.ds(g * LN, LN)] = sdP[1, pl.ds(g * LN, LN)]
            plsc.store_scatter(waP, [lane + g * LN, zcol],
                               w_v[pl.ds(g * LN, LN)])
        for g in range(HB // LN):
            dstB[pl.ds(g * LN, LN)] = sdP[1, pl.ds(HA + g * LN, LN)]
            plsc.store_scatter(wrowB, [lane + g * LN, zcol],
                               w_v[pl.ds(HA + g * LN, LN)])

        pltpu.async_copy(z_hbm.at[sdP.at[0, pl.ds(HA, HB)]], zrowB, gsemB)
        pltpu.make_async_copy(z_hbm.at[sdP.at[0, pl.ds(0, HA)]],
                              zaP, gsemA).wait()

        scale_rows(zaP, HA, 0)
        pltpu.async_copy(zaP, acch_sh.at[daP], hsemP, add=True)
        pltpu.async_copy(waP, accd_sh.at[daP], dsemP, add=True)

        pltpu.make_async_copy(z_hbm.at[sdP.at[0, pl.ds(HA, HB)]],
                              zrowB, gsemB).wait()
        scale_rows(zrowB, HB, HA)

        def drain_prev_a():
            pltpu.make_async_copy(zaQ, acch_sh.at[daQ], hsemQ).wait()
            pltpu.make_async_copy(waQ, accd_sh.at[daQ], dsemQ).wait()

        if prev_pred is None:
            drain_prev_a()
        else:
            pl.when(prev_pred)(drain_prev_a)
        if has_next:
            pltpu.make_async_copy(sd_hbm.at[cbase + ci + 1], sdQ, isem).wait()
            pltpu.async_copy(z_hbm.at[sdQ.at[0, pl.ds(0, HA)]], zaQ, gsemA)

        pltpu.async_copy(zrowB, acch_sh.at[dstB], hsemB, add=True)
        pltpu.async_copy(wrowB, accd_sh.at[dstB], dsemB, add=True)

    pltpu.sync_copy(sd_hbm.at[cbase], sdA)
    pltpu.async_copy(z_hbm.at[sdA.at[0, pl.ds(0, HA)]], zA1, gsemA)

    @pl.loop(0, NCHUNK // 2)
    def _pair(k):
        chunk_body(2 * k, sdA, sdB, zA1, wA1, dA1, hA1sem, dA1sem,
                   zA2, wA2, dA2, hA2sem, dA2sem, k > 0, True)
        chunk_body(2 * k + 1, sdB, sdA, zA2, wA2, dA2, hA2sem, dA2sem,
                   zA1, wA1, dA1, hA1sem, dA1sem, None, True)

    chunk_body(NCHUNK - 1, sdA, sdB, zA1, wA1, dA1, hA1sem, dA1sem,
               zA2, wA2, dA2, hA2sem, dA2sem, None, False)
    pltpu.make_async_copy(zA1, acch_sh.at[dA1], hA1sem).wait()
    pltpu.make_async_copy(wA1, accd_sh.at[dA1], dA1sem).wait()
    pltpu.make_async_copy(zrowB, acch_sh.at[dstB], hsemB).wait()
    pltpu.make_async_copy(wrowB, accd_sh.at[dstB], dsemB).wait()

    plsc.subcore_barrier()

    @pl.loop(sid, N // CH, step=NS)
    def _out(g):
        pltpu.sync_copy(acch_sh.at[pl.ds(g * CH, CH)],
                        outh_hbm.at[cid, pl.ds(g * CH, CH)])
        pltpu.sync_copy(accd_sh.at[pl.ds(g * CH, CH)],
                        outd_hbm.at[cid, pl.ds(g * CH, CH)])


def kernel(x, edge_index, W_gat, attn_l, attn_r, bias_gat, W_lin):
    ei = edge_index.astype(jnp.int32)
    sd = ei.reshape(2, E // CH, CH).transpose(1, 0, 2)

    z, el, er, m = pl.pallas_call(
        _pre_body,
        out_shape=[
            jax.ShapeDtypeStruct((N, D), jnp.float32),
            jax.ShapeDtypeStruct((N,), jnp.float32),
            jax.ShapeDtypeStruct((N,), jnp.float32),
            jax.ShapeDtypeStruct((LN,), jnp.float32),
        ],
    )(x, W_gat, attn_l, attn_r)

    mesh = plsc.VectorSubcoreMesh(core_axis_name="c", subcore_axis_name="s",
                                  num_cores=NC, num_subcores=NS)
    edge_kernel = pl.kernel(
        _edge_body,
        out_type=[
            jax.ShapeDtypeStruct((NC, N, D), jnp.float32),
            jax.ShapeDtypeStruct((NC, N, LN), jnp.float32),
        ],
        mesh=mesh,
        compiler_params=pltpu.CompilerParams(use_tc_tiling_on_sc=False,
                                             needs_layout_passes=False),
        scratch_types=[
            pltpu.VMEM((N,), jnp.float32),
            pltpu.VMEM((N,), jnp.float32),
            pltpu.VMEM((LN,), jnp.float32),
            pltpu.VMEM((2, CH), jnp.int32),
            pltpu.VMEM((2, CH), jnp.int32),
            pltpu.VMEM((HA, D), jnp.float32),
            pltpu.VMEM((HA, D), jnp.float32),
            pltpu.VMEM((HB, D), jnp.float32),
            pltpu.VMEM((HA, LN), jnp.float32),
            pltpu.VMEM((HA, LN), jnp.float32),
            pltpu.VMEM((HB, LN), jnp.float32),
            pltpu.VMEM((HA,), jnp.int32),
            pltpu.VMEM((HA,), jnp.int32),
            pltpu.VMEM((HB,), jnp.int32),
            pltpu.VMEM((CH + LN,), jnp.float32),
            pltpu.VMEM_SHARED((N, D), jnp.float32),
            pltpu.VMEM_SHARED((N, LN), jnp.float32),
            pltpu.SemaphoreType.DMA,
            pltpu.SemaphoreType.DMA,
            pltpu.SemaphoreType.DMA,
            pltpu.SemaphoreType.DMA,
            pltpu.SemaphoreType.DMA,
            pltpu.SemaphoreType.DMA,
            pltpu.SemaphoreType.DMA,
            pltpu.SemaphoreType.DMA,
            pltpu.SemaphoreType.DMA,
        ],
    )
    parts_h, parts_d = edge_kernel(z, el, er, sd, m)

    out = pl.pallas_call(
        _post_body,
        out_shape=jax.ShapeDtypeStruct((N, D), jnp.float32),
    )(parts_h, parts_d, bias_gat, W_lin)
    return out
